# baseline (speedup 1.0000x reference)
import functools
import jax
import jax.numpy as jnp
from jax import lax
from jax.experimental import pallas as pl
from jax.experimental.pallas import tpu as pltpu
from jax.experimental.pallas import tpu_sc as plsc

_R, _C = 4096, 2048
_NC, _NS = 2, 16
_NW = _NC * _NS
_TOTAL = _R * _C
_PER_W = _TOTAL // _NW
_CHUNK = 16384
_NCHUNK = _PER_W // _CHUNK
_NVREG = _CHUNK // 16


def _sc_body(x_hbm, o_hbm, bufs_in, bufs_out, sems_in, sems_out):
    wid = lax.axis_index("s") * _NC + lax.axis_index("c")
    base = wid * _PER_W
    lane = lax.iota(jnp.int32, 16)
    idx_swap = lane ^ 1
    even = (lane % 2) == 0

    def off(k):
        return base + k * _CHUNK

    def compute(slot):
        def vbody(j, carry):
            v = bufs_in[slot, pl.ds(j * 16, 16)]
            p = v[idx_swap]
            lo = jnp.minimum(v, p)
            hi = jnp.maximum(v, p)
            bufs_out[slot, pl.ds(j * 16, 16)] = jnp.where(even, lo, hi)
            return carry
        lax.fori_loop(0, _NVREG, vbody, 0, unroll=8)

    pltpu.make_async_copy(
        x_hbm.at[pl.ds(off(0), _CHUNK)], bufs_in.at[0], sems_in.at[0]
    ).start()

    def body(k, carry):
        slot = lax.rem(k, 2)
        nxt = lax.rem(k + 1, 2)

        @pl.when(k + 1 < _NCHUNK)
        def _():
            pltpu.make_async_copy(
                x_hbm.at[pl.ds(off(k + 1), _CHUNK)], bufs_in.at[nxt],
                sems_in.at[nxt],
            ).start()

        pltpu.make_async_copy(
            x_hbm.at[pl.ds(off(k), _CHUNK)], bufs_in.at[slot], sems_in.at[slot]
        ).wait()

        @pl.when(k >= 2)
        def _():
            pltpu.make_async_copy(
                bufs_out.at[slot], o_hbm.at[pl.ds(off(k - 2), _CHUNK)],
                sems_out.at[slot],
            ).wait()

        compute(slot)

        pltpu.make_async_copy(
            bufs_out.at[slot], o_hbm.at[pl.ds(off(k), _CHUNK)], sems_out.at[slot]
        ).start()
        return carry

    lax.fori_loop(0, _NCHUNK, body, 0)

    pltpu.make_async_copy(
        bufs_out.at[_NCHUNK % 2], o_hbm.at[pl.ds(off(_NCHUNK - 2), _CHUNK)],
        sems_out.at[_NCHUNK % 2],
    ).wait()
    pltpu.make_async_copy(
        bufs_out.at[(_NCHUNK - 1) % 2], o_hbm.at[pl.ds(off(_NCHUNK - 1), _CHUNK)],
        sems_out.at[(_NCHUNK - 1) % 2],
    ).wait()


@jax.jit
def _twosort_sc(x_flat):
    mesh = plsc.VectorSubcoreMesh(core_axis_name="c", subcore_axis_name="s")
    return pl.kernel(
        _sc_body,
        out_type=jax.ShapeDtypeStruct((_TOTAL,), jnp.float32),
        mesh=mesh,
        scratch_types=[
            pltpu.VMEM((2, _CHUNK), jnp.float32),
            pltpu.VMEM((2, _CHUNK), jnp.float32),
            pltpu.SemaphoreType.DMA((2,)),
            pltpu.SemaphoreType.DMA((2,)),
        ],
    )(x_flat)


def kernel(x):
    return _twosort_sc(x.reshape(-1)).reshape(x.shape)


# speedup vs baseline: 1.6940x; 1.6940x over previous
import functools
import jax
import jax.numpy as jnp
from jax import lax
from jax.experimental import pallas as pl
from jax.experimental.pallas import tpu as pltpu
from jax.experimental.pallas import tpu_sc as plsc

_R, _C = 4096, 2048
_NC, _NS = 2, 16
_NW = _NC * _NS
_TOTAL = _R * _C
_PER_W = _TOTAL // _NW
_CHUNK = 16384
_NCHUNK = _PER_W // _CHUNK
_NVREG = _CHUNK // 16


def _sc_body(x_hbm, o_hbm, bufs_in, bufs_out, sems_in, sems_out):
    wid = lax.axis_index("s") * _NC + lax.axis_index("c")
    base = wid * _PER_W
    lane = lax.iota(jnp.int32, 16)
    idx_swap = lane ^ 1
    even = (lane % 2) == 0

    def off(k):
        return base + k * _CHUNK

    def compute(slot):
        @plsc.parallel_loop(0, _CHUNK, step=16, unroll=8)
        def _(i):
            v = bufs_in[slot, pl.ds(i, 16)]
            p = v[idx_swap]
            lo = jnp.minimum(v, p)
            hi = jnp.maximum(v, p)
            bufs_out[slot, pl.ds(i, 16)] = jnp.where(even, lo, hi)

    pltpu.make_async_copy(
        x_hbm.at[pl.ds(off(0), _CHUNK)], bufs_in.at[0], sems_in.at[0]
    ).start()

    def body(k, carry):
        slot = lax.rem(k, 2)
        nxt = lax.rem(k + 1, 2)

        @pl.when(k + 1 < _NCHUNK)
        def _():
            pltpu.make_async_copy(
                x_hbm.at[pl.ds(off(k + 1), _CHUNK)], bufs_in.at[nxt],
                sems_in.at[nxt],
            ).start()

        pltpu.make_async_copy(
            x_hbm.at[pl.ds(off(k), _CHUNK)], bufs_in.at[slot], sems_in.at[slot]
        ).wait()

        @pl.when(k >= 2)
        def _():
            pltpu.make_async_copy(
                bufs_out.at[slot], o_hbm.at[pl.ds(off(k - 2), _CHUNK)],
                sems_out.at[slot],
            ).wait()

        compute(slot)

        pltpu.make_async_copy(
            bufs_out.at[slot], o_hbm.at[pl.ds(off(k), _CHUNK)], sems_out.at[slot]
        ).start()
        return carry

    lax.fori_loop(0, _NCHUNK, body, 0)

    pltpu.make_async_copy(
        bufs_out.at[_NCHUNK % 2], o_hbm.at[pl.ds(off(_NCHUNK - 2), _CHUNK)],
        sems_out.at[_NCHUNK % 2],
    ).wait()
    pltpu.make_async_copy(
        bufs_out.at[(_NCHUNK - 1) % 2], o_hbm.at[pl.ds(off(_NCHUNK - 1), _CHUNK)],
        sems_out.at[(_NCHUNK - 1) % 2],
    ).wait()


@jax.jit
def _twosort_sc(x_flat):
    mesh = plsc.VectorSubcoreMesh(core_axis_name="c", subcore_axis_name="s")
    return pl.kernel(
        _sc_body,
        out_type=jax.ShapeDtypeStruct((_TOTAL,), jnp.float32),
        mesh=mesh,
        scratch_types=[
            pltpu.VMEM((2, _CHUNK), jnp.float32),
            pltpu.VMEM((2, _CHUNK), jnp.float32),
            pltpu.SemaphoreType.DMA((2,)),
            pltpu.SemaphoreType.DMA((2,)),
        ],
    )(x_flat)


def kernel(x):
    return _twosort_sc(x.reshape(-1)).reshape(x.shape)


# speedup vs baseline: 4.3184x; 2.5492x over previous
import functools
import jax
import jax.numpy as jnp
from jax import lax
from jax.experimental import pallas as pl
from jax.experimental.pallas import tpu as pltpu
from jax.experimental.pallas import tpu_sc as plsc

_R, _C = 4096, 2048
_NC, _NS = 2, 16
_NW = _NC * _NS
_RPW = _R // _NW
_CR = 8
_NCHUNK = _RPW // _CR


def _sc_body(x_hbm, o_hbm, bufs_in, bufs_out, sems_in, sems_out):
    wid = lax.axis_index("s") * _NC + lax.axis_index("c")
    base_row = wid * _RPW
    lane = lax.iota(jnp.int32, 16)
    idx_swap = lane ^ 1
    even = (lane % 2) == 0

    def row0(k):
        return base_row + k * _CR

    def compute(slot):
        for r in range(_CR):
            @plsc.parallel_loop(0, _C, step=16, unroll=8)
            def _(i):
                v = bufs_in[slot, r, pl.ds(i, 16)]
                p = v[idx_swap]
                lo = jnp.minimum(v, p)
                hi = jnp.maximum(v, p)
                bufs_out[slot, r, pl.ds(i, 16)] = jnp.where(even, lo, hi)

    pltpu.make_async_copy(
        x_hbm.at[pl.ds(row0(0), _CR), :], bufs_in.at[0], sems_in.at[0]
    ).start()

    def body(k, carry):
        slot = lax.rem(k, 2)
        nxt = lax.rem(k + 1, 2)

        @pl.when(k + 1 < _NCHUNK)
        def _():
            pltpu.make_async_copy(
                x_hbm.at[pl.ds(row0(k + 1), _CR), :], bufs_in.at[nxt],
                sems_in.at[nxt],
            ).start()

        pltpu.make_async_copy(
            x_hbm.at[pl.ds(row0(k), _CR), :], bufs_in.at[slot], sems_in.at[slot]
        ).wait()

        @pl.when(k >= 2)
        def _():
            pltpu.make_async_copy(
                bufs_out.at[slot], o_hbm.at[pl.ds(row0(k - 2), _CR), :],
                sems_out.at[slot],
            ).wait()

        compute(slot)

        pltpu.make_async_copy(
            bufs_out.at[slot], o_hbm.at[pl.ds(row0(k), _CR), :], sems_out.at[slot]
        ).start()
        return carry

    lax.fori_loop(0, _NCHUNK, body, 0)

    pltpu.make_async_copy(
        bufs_out.at[_NCHUNK % 2], o_hbm.at[pl.ds(row0(_NCHUNK - 2), _CR), :],
        sems_out.at[_NCHUNK % 2],
    ).wait()
    pltpu.make_async_copy(
        bufs_out.at[(_NCHUNK - 1) % 2],
        o_hbm.at[pl.ds(row0(_NCHUNK - 1), _CR), :],
        sems_out.at[(_NCHUNK - 1) % 2],
    ).wait()


@jax.jit
def _twosort_sc(x):
    mesh = plsc.VectorSubcoreMesh(core_axis_name="c", subcore_axis_name="s")
    return pl.kernel(
        _sc_body,
        out_type=jax.ShapeDtypeStruct((_R, _C), jnp.float32),
        mesh=mesh,
        scratch_types=[
            pltpu.VMEM((2, _CR, _C), jnp.float32),
            pltpu.VMEM((2, _CR, _C), jnp.float32),
            pltpu.SemaphoreType.DMA((2,)),
            pltpu.SemaphoreType.DMA((2,)),
        ],
    )(x)


def kernel(x):
    return _twosort_sc(x)
